# baseline (speedup 1.0000x reference)
import jax
import jax.numpy as jnp
from jax.experimental import pallas as pl
from jax.experimental.pallas import tpu as pltpu

_TILE = 1024


def _cs_kernel(labels_ref, feats_ref, w_ref, b_ref, queue_ref, out_ref):
    feats = feats_ref[0]
    proj = jnp.dot(w_ref[...], feats, preferred_element_type=jnp.float32)
    proj = proj + b_ref[...]
    q = queue_ref[0]
    logit = jnp.dot(q, proj, preferred_element_type=jnp.float32)
    m = jnp.max(logit, axis=0, keepdims=True)
    e = jnp.exp(logit - m)
    p = e / jnp.sum(e, axis=0, keepdims=True)
    new = jnp.dot(q.T, p, preferred_element_type=jnp.float32)
    code = new.shape[0]
    out_ref[0, :code, :] = new
    out_ref[0, code:, :] = proj


@jax.jit
def _run(feats, labels, W_proj, b_proj, queue):
    B, C, H, W = feats.shape
    HW = H * W
    code = W_proj.shape[0]
    feats3 = feats.reshape(B, C, HW)
    nt = HW // _TILE
    grid_spec = pltpu.PrefetchScalarGridSpec(
        num_scalar_prefetch=1,
        grid=(B, nt),
        in_specs=[
            pl.BlockSpec((1, C, _TILE), lambda b, j, lbl: (b, 0, j)),
            pl.BlockSpec((code, C), lambda b, j, lbl: (0, 0)),
            pl.BlockSpec((code, 1), lambda b, j, lbl: (0, 0)),
            pl.BlockSpec((1,) + queue.shape[1:], lambda b, j, lbl: (lbl[b], 0, 0)),
        ],
        out_specs=pl.BlockSpec((1, 2 * code, _TILE), lambda b, j, lbl: (b, 0, j)),
    )
    out = pl.pallas_call(
        _cs_kernel,
        grid_spec=grid_spec,
        out_shape=jax.ShapeDtypeStruct((B, 2 * code, HW), jnp.float32),
        compiler_params=pltpu.CompilerParams(
            dimension_semantics=("parallel", "parallel"),
        ),
    )(labels.astype(jnp.int32), feats3, W_proj, b_proj.reshape(code, 1), queue)
    return out.reshape(B, 2 * code, H, W)


def kernel(feats, preds, labels, flag, W_proj, b_proj, queue):
    return _run(feats, labels, W_proj, b_proj, queue)


# speedup vs baseline: 1.0700x; 1.0700x over previous
import jax
import jax.numpy as jnp
from jax.experimental import pallas as pl
from jax.experimental.pallas import tpu as pltpu

_TILE = 4096


def _cs_kernel(labels_ref, feats_ref, w_ref, b_ref, queue_ref, out_ref):
    feats = feats_ref[0]
    proj = jnp.dot(w_ref[...], feats, preferred_element_type=jnp.float32)
    proj = proj + b_ref[...]
    q = queue_ref[0]
    logit = jnp.dot(q, proj, preferred_element_type=jnp.float32)
    m = jnp.max(logit, axis=0, keepdims=True)
    e = jnp.exp(logit - m)
    p = e / jnp.sum(e, axis=0, keepdims=True)
    new = jnp.dot(q.T, p, preferred_element_type=jnp.float32)
    code = new.shape[0]
    out_ref[0, :code, :] = new
    out_ref[0, code:, :] = proj


@jax.jit
def _run(feats, labels, W_proj, b_proj, queue):
    B, C, H, W = feats.shape
    HW = H * W
    code = W_proj.shape[0]
    feats3 = feats.reshape(B, C, HW)
    nt = HW // _TILE
    grid_spec = pltpu.PrefetchScalarGridSpec(
        num_scalar_prefetch=1,
        grid=(B, nt),
        in_specs=[
            pl.BlockSpec((1, C, _TILE), lambda b, j, lbl: (b, 0, j)),
            pl.BlockSpec((code, C), lambda b, j, lbl: (0, 0)),
            pl.BlockSpec((code, 1), lambda b, j, lbl: (0, 0)),
            pl.BlockSpec((1,) + queue.shape[1:], lambda b, j, lbl: (lbl[b], 0, 0)),
        ],
        out_specs=pl.BlockSpec((1, 2 * code, _TILE), lambda b, j, lbl: (b, 0, j)),
    )
    out = pl.pallas_call(
        _cs_kernel,
        grid_spec=grid_spec,
        out_shape=jax.ShapeDtypeStruct((B, 2 * code, HW), jnp.float32),
        compiler_params=pltpu.CompilerParams(
            dimension_semantics=("parallel", "parallel"),
        ),
    )(labels.astype(jnp.int32), feats3, W_proj, b_proj.reshape(code, 1), queue)
    return out.reshape(B, 2 * code, H, W)


def kernel(feats, preds, labels, flag, W_proj, b_proj, queue):
    return _run(feats, labels, W_proj, b_proj, queue)
